# Initial kernel scaffold; baseline (speedup 1.0000x reference)
import jax
import jax.numpy as jnp
from jax.experimental import pallas as pl


def kernel(x, edge_index, batch, atom_emb, anchor_emb, gin_W1, gin_b1, gin_W2, gin_b2, gin_eps, n2n_W, n2n_b, dl_W, dl_b, pl_W, pl_b):
    raise NotImplementedError("write your pallas kernel here")


# speedup vs baseline: 1.0431x; 1.0431x over previous
import functools

import jax
import jax.numpy as jnp
from jax.experimental import pallas as pl
from jax.experimental.pallas import tpu as pltpu

_N = 10000
_E = 320000
_G = 64
_D = 128
_L = 5
_BLK = 1000


def _gin_dense_body(scale_ref, h_ref, agg_ref, w1_ref, b1_ref, w2_ref, b2_ref, out_ref):
    z = scale_ref[0] * h_ref[...] + agg_ref[...]
    z = jnp.maximum(
        jnp.dot(z, w1_ref[...], preferred_element_type=jnp.float32) + b1_ref[...], 0.0)
    out_ref[...] = jnp.maximum(
        jnp.dot(z, w2_ref[...], preferred_element_type=jnp.float32) + b2_ref[...], 0.0)


@jax.jit
def _gin_dense(scale, h, agg, w1, b1, w2, b2):
    return pl.pallas_call(
        _gin_dense_body,
        grid=(_N // _BLK,),
        in_specs=[
            pl.BlockSpec(memory_space=pltpu.SMEM),
            pl.BlockSpec((_BLK, _D), lambda i: (i, 0)),
            pl.BlockSpec((_BLK, _D), lambda i: (i, 0)),
            pl.BlockSpec((_D, _D), lambda i: (0, 0)),
            pl.BlockSpec((1, _D), lambda i: (0, 0)),
            pl.BlockSpec((_D, _D), lambda i: (0, 0)),
            pl.BlockSpec((1, _D), lambda i: (0, 0)),
        ],
        out_specs=pl.BlockSpec((_BLK, _D), lambda i: (i, 0)),
        out_shape=jax.ShapeDtypeStruct((_N, _D), jnp.float32),
    )(scale, h, agg, w1, b1.reshape(1, _D), w2, b2.reshape(1, _D))


def _gnn(h, src, dst, gin_W1, gin_b1, gin_W2, gin_b2, gin_eps):
    for l in range(_L):
        agg = jax.ops.segment_sum(h[src], dst, num_segments=_N)
        h = _gin_dense(jnp.reshape(1.0 + gin_eps[l], (1,)), h, agg,
                       gin_W1[l], gin_b1[l], gin_W2[l], gin_b2[l])
    return h


def kernel(x, edge_index, batch, atom_emb, anchor_emb, gin_W1, gin_b1,
           gin_W2, gin_b2, gin_eps, n2n_W, n2n_b, dl_W, dl_b, pl_W, pl_b):
    x = x.astype(jnp.int32)
    src = edge_index[0].astype(jnp.int32)
    dst = edge_index[1].astype(jnp.int32)
    n = _N

    anchor_label = jnp.zeros((n,), dtype=jnp.int32)
    u = jax.random.uniform(jax.random.fold_in(jax.random.key(12345), 0),
                           (n,), minval=1e-9, maxval=1.0)
    gumb = -jnp.log(-jnp.log(u))

    h0 = atom_emb[x] + anchor_emb[anchor_label]
    h = _gnn(h0, src, dst, gin_W1, gin_b1, gin_W2, gin_b2, gin_eps)
    pred = (h @ dl_W + dl_b)[:, 0]
    m = jax.ops.segment_max(pred, batch, num_segments=_G)
    e = jnp.exp(pred - m[batch])
    s = jax.ops.segment_sum(e, batch, num_segments=_G)
    prob = e / (s[batch] + 1e-15)
    z = jnp.log(prob + 1e-15) + gumb
    zmax = jax.ops.segment_max(z, batch, num_segments=_G)
    cand = jnp.where(z >= zmax[batch] - 1e-6, jnp.arange(n), -1)
    rawsample = jnp.maximum(jax.ops.segment_max(cand, batch, num_segments=_G), 0)
    anchor_label = anchor_label.at[rawsample].set(1)

    h0 = atom_emb[x] + anchor_emb[anchor_label]
    h = _gnn(h0, src, dst, gin_W1, gin_b1, gin_W2, gin_b2, gin_eps)
    h2 = jax.nn.relu(h @ n2n_W + n2n_b)
    counts = jax.ops.segment_sum(jnp.ones((n,), dtype=jnp.float32), batch,
                                 num_segments=_G)
    pooled = jax.ops.segment_sum(h2, batch, num_segments=_G) / jnp.maximum(
        counts, 1.0)[:, None]
    return pooled @ pl_W + pl_b
